# baseline (speedup 1.0000x reference)
import jax
import jax.numpy as jnp
from jax import lax
from jax.experimental import pallas as pl
from jax.experimental.pallas import tpu as pltpu
from jax.experimental.pallas import tpu_sc as plsc

H = 512
W = 512
B = 16
P = H * W
R = 10.0
MAGIC = 1.5 * (2.0 ** 23)

NC = 2
NS = 16
L = 16

BATCHES_PER_CORE = B // NC
PASS_BATCHES = 4
NPASS = BATCHES_PER_CORE // PASS_BATCHES
TILES_PER_BATCH = NS // PASS_BATCHES
ROWS_PER_TILE = H // TILES_PER_BATCH
RCH = 8
CH = RCH * W
NCHUNK = ROWS_PER_TILE // RCH
SLOTS = 3
HIST = PASS_BATCHES * P
SLICE = HIST // NS
ZB = 2048


def _body(w_hbm, off_hbm, out_hbm, hist,
          dx_a, dx_b, dx_c, dy_a, dy_b, dy_c, w_a, w_b, w_c,
          w1_a, w1_b, w1_c, idx_a, idx_b, idx_c, zero_v, xf_buf,
          sin_a, sin_b, sin_c, ssc_a, ssc_b, ssc_c):
    c = lax.axis_index("c")
    s = lax.axis_index("s")
    b_in_pass = s // TILES_PER_BATCH
    part = s % TILES_PER_BATCH
    row_base = part * ROWS_PER_TILE
    myslice = s * SLICE
    lanes = lax.iota(jnp.int32, L)

    dx_r = (dx_a, dx_b, dx_c)
    dy_r = (dy_a, dy_b, dy_c)
    w_r = (w_a, w_b, w_c)
    w1_r = (w1_a, w1_b, w1_c)
    idx_r = (idx_a, idx_b, idx_c)
    sin_r = (sin_a, sin_b, sin_c)
    ssc_r = (ssc_a, ssc_b, ssc_c)

    def zinit(i, carry):
        zero_v[pl.ds(i * L, L)] = jnp.zeros((L,), jnp.float32)
        return carry

    lax.fori_loop(0, ZB // L, zinit, 0)

    def xinit(j, carry):
        xf_buf[pl.ds(j * L, L)] = (lanes + j * L).astype(jnp.float32)
        return carry

    lax.fori_loop(0, W // L, xinit, 0)

    def pass_body(pidx, pcarry):
        b_global = c * BATCHES_PER_CORE + pidx * PASS_BATCHES + b_in_pass
        for q in range(SLICE // ZB):
            pltpu.sync_copy(zero_v, hist.at[pl.ds(myslice + q * ZB, ZB)])
        plsc.subcore_barrier()

        hist_off = b_in_pass * P

        def fire_inputs(ci, sl):
            y0 = row_base + ci * RCH
            pltpu.async_copy(
                w_hbm.at[b_global, pl.ds(y0, RCH), :], w_r[sl], sin_r[sl])
            pltpu.async_copy(
                off_hbm.at[2 * b_global, pl.ds(y0, RCH), :], dx_r[sl],
                sin_r[sl])
            pltpu.async_copy(
                off_hbm.at[2 * b_global + 1, pl.ds(y0, RCH), :], dy_r[sl],
                sin_r[sl])

        def wait_inputs(sl):
            src = w_hbm.at[b_global, pl.ds(0, RCH), :]
            pltpu.make_async_copy(src, w_r[sl], sin_r[sl]).wait()
            pltpu.make_async_copy(src, dx_r[sl], sin_r[sl]).wait()
            pltpu.make_async_copy(src, dy_r[sl], sin_r[sl]).wait()

        def wait_scatter(sl):
            pltpu.make_async_copy(
                w1_r[sl], hist.at[idx_r[sl]], ssc_r[sl]).wait()

        fire_inputs(0, 0)

        def chunk_body(ci, carry):
            for k in range(SLOTS):

                @pl.when(ci % SLOTS == k)
                def _process(k=k):
                    nxt = (k + 1) % SLOTS

                    @pl.when(ci + 1 < NCHUNK)
                    def _prefetch():
                        @pl.when(ci >= 2)
                        def _drain():
                            wait_scatter(nxt)

                        fire_inputs(ci + 1, nxt)

                    wait_inputs(k)
                    y0 = row_base + ci * RCH
                    dx_cur = dx_r[k]
                    dy_cur = dy_r[k]
                    w_cur = w_r[k]
                    w1_cur = w1_r[k]
                    idx_cur = idx_r[k]
                    yfs = [(y0 + r).astype(jnp.float32) for r in range(RCH)]

                    def vec_body(j, rcarry):
                        xf = xf_buf[pl.ds(j * L, L)]
                        for r in range(RCH):
                            dxv = dx_cur[r, pl.ds(j * L, L)]
                            dyv = dy_cur[r, pl.ds(j * L, L)]
                            wv = w_cur[r, pl.ds(j * L, L)]
                            vx = (xf + R * dxv + MAGIC) - MAGIC
                            vy = (yfs[r] + R * dyv + MAGIC) - MAGIC
                            vx = jnp.minimum(jnp.maximum(vx, 0.0), W - 1.0)
                            vy = jnp.minimum(jnp.maximum(vy, 0.0), H - 1.0)
                            idx = (lax.shift_left(vy.astype(jnp.int32), 9)
                                   + vx.astype(jnp.int32) + hist_off)
                            o = r * W + j * L
                            idx_cur[pl.ds(o, L)] = idx
                            w1_cur[pl.ds(o, L)] = wv
                        return rcarry

                    lax.fori_loop(0, W // L, vec_body, 0)
                    pltpu.async_copy(
                        w1_cur, hist.at[idx_cur], ssc_r[k], add=True)

            return carry

        lax.fori_loop(0, NCHUNK, chunk_body, 0)
        for sl in range(SLOTS):
            wait_scatter(sl)
        plsc.subcore_barrier()

        pass_out = (c * BATCHES_PER_CORE + pidx * PASS_BATCHES) * P
        pltpu.sync_copy(hist.at[pl.ds(myslice, SLICE)],
                        out_hbm.at[pl.ds(pass_out + myslice, SLICE)])
        return pcarry

    lax.fori_loop(0, NPASS, pass_body, 0)


def kernel(stem_keypoint_output, stem_offset_output):
    w3 = stem_keypoint_output.reshape(B, H, W)
    off3 = stem_offset_output.reshape(2 * B, H, W)
    mesh = plsc.VectorSubcoreMesh(core_axis_name="c", subcore_axis_name="s")
    out = pl.kernel(
        _body,
        out_type=jax.ShapeDtypeStruct((B * P,), jnp.float32),
        mesh=mesh,
        compiler_params=pltpu.CompilerParams(use_tc_tiling_on_sc=True),
        scratch_types=[
            pltpu.VMEM_SHARED((HIST,), jnp.float32),
            pltpu.VMEM((RCH, W), jnp.float32),
            pltpu.VMEM((RCH, W), jnp.float32),
            pltpu.VMEM((RCH, W), jnp.float32),
            pltpu.VMEM((RCH, W), jnp.float32),
            pltpu.VMEM((RCH, W), jnp.float32),
            pltpu.VMEM((RCH, W), jnp.float32),
            pltpu.VMEM((RCH, W), jnp.float32),
            pltpu.VMEM((RCH, W), jnp.float32),
            pltpu.VMEM((RCH, W), jnp.float32),
            pltpu.VMEM((CH,), jnp.float32),
            pltpu.VMEM((CH,), jnp.float32),
            pltpu.VMEM((CH,), jnp.float32),
            pltpu.VMEM((CH,), jnp.int32),
            pltpu.VMEM((CH,), jnp.int32),
            pltpu.VMEM((CH,), jnp.int32),
            pltpu.VMEM((ZB,), jnp.float32),
            pltpu.VMEM((W,), jnp.float32),
            pltpu.SemaphoreType.DMA,
            pltpu.SemaphoreType.DMA,
            pltpu.SemaphoreType.DMA,
            pltpu.SemaphoreType.DMA,
            pltpu.SemaphoreType.DMA,
            pltpu.SemaphoreType.DMA,
        ],
    )(w3, off3)
    return out.reshape(B, H, W)


# speedup vs baseline: 1.0359x; 1.0359x over previous
import jax
import jax.numpy as jnp
from jax import lax
from jax.experimental import pallas as pl
from jax.experimental.pallas import tpu as pltpu
from jax.experimental.pallas import tpu_sc as plsc

H = 512
W = 512
B = 16
P = H * W
R = 10.0
MAGIC = 1.5 * (2.0 ** 23)

NC = 2
NS = 16
L = 16

BATCHES_PER_CORE = B // NC
PASS_BATCHES = 4
NPASS = BATCHES_PER_CORE // PASS_BATCHES
TILES_PER_BATCH = NS // PASS_BATCHES
ROWS_PER_TILE = H // TILES_PER_BATCH
RCH = 8
CH = RCH * W
NCHUNK = ROWS_PER_TILE // RCH
SLOTS = 3
HIST = PASS_BATCHES * P
SLICE = HIST // NS
ZB = 2048


def _body(w_hbm, off_hbm, out_hbm, hist,
          dx_a, dx_b, dx_c, dy_a, dy_b, dy_c, w_a, w_b, w_c,
          w1_a, w1_b, w1_c, idx_a, idx_b, idx_c, zero_v, xf_buf,
          sin_a, sin_b, sin_c, ssc_a, ssc_b, ssc_c):
    c = lax.axis_index("c")
    s = lax.axis_index("s")
    b_in_pass = s // TILES_PER_BATCH
    part = s % TILES_PER_BATCH
    row_base = part * ROWS_PER_TILE
    myslice = s * SLICE
    lanes = lax.iota(jnp.int32, L)

    dx_r = (dx_a, dx_b, dx_c)
    dy_r = (dy_a, dy_b, dy_c)
    w_r = (w_a, w_b, w_c)
    w1_r = (w1_a, w1_b, w1_c)
    idx_r = (idx_a, idx_b, idx_c)
    sin_r = (sin_a, sin_b, sin_c)
    ssc_r = (ssc_a, ssc_b, ssc_c)

    def zinit(i, carry):
        zero_v[pl.ds(i * L, L)] = jnp.zeros((L,), jnp.float32)
        return carry

    lax.fori_loop(0, ZB // L, zinit, 0)

    def xinit(j, carry):
        xf_buf[pl.ds(j * L, L)] = (lanes + j * L).astype(jnp.float32)
        return carry

    lax.fori_loop(0, W // L, xinit, 0)

    def fire_first_inputs():
        bg0 = c * BATCHES_PER_CORE + b_in_pass
        y0 = row_base
        pltpu.async_copy(w_hbm.at[bg0, pl.ds(y0, RCH), :], w_a, sin_a)
        pltpu.async_copy(off_hbm.at[2 * bg0, pl.ds(y0, RCH), :], dx_a, sin_a)
        pltpu.async_copy(off_hbm.at[2 * bg0 + 1, pl.ds(y0, RCH), :], dy_a,
                         sin_a)

    fire_first_inputs()

    def pass_body(pidx, pcarry):
        b_global = c * BATCHES_PER_CORE + pidx * PASS_BATCHES + b_in_pass
        for q in range(SLICE // ZB):
            pltpu.async_copy(zero_v, hist.at[pl.ds(myslice + q * ZB, ZB)],
                             ssc_a)
        for q in range(SLICE // ZB):
            pltpu.make_async_copy(
                zero_v, hist.at[pl.ds(myslice + q * ZB, ZB)], ssc_a).wait()
        plsc.subcore_barrier()

        hist_off = b_in_pass * P

        def fire_inputs(ci, sl, bg=None):
            bg = b_global if bg is None else bg
            y0 = row_base + ci * RCH
            pltpu.async_copy(
                w_hbm.at[bg, pl.ds(y0, RCH), :], w_r[sl], sin_r[sl])
            pltpu.async_copy(
                off_hbm.at[2 * bg, pl.ds(y0, RCH), :], dx_r[sl],
                sin_r[sl])
            pltpu.async_copy(
                off_hbm.at[2 * bg + 1, pl.ds(y0, RCH), :], dy_r[sl],
                sin_r[sl])

        def wait_inputs(sl):
            src = w_hbm.at[b_global, pl.ds(0, RCH), :]
            pltpu.make_async_copy(src, w_r[sl], sin_r[sl]).wait()
            pltpu.make_async_copy(src, dx_r[sl], sin_r[sl]).wait()
            pltpu.make_async_copy(src, dy_r[sl], sin_r[sl]).wait()

        def wait_scatter(sl):
            pltpu.make_async_copy(
                w1_r[sl], hist.at[idx_r[sl]], ssc_r[sl]).wait()

        def chunk_body(ci, carry):
            for k in range(SLOTS):

                @pl.when(ci % SLOTS == k)
                def _process(k=k):
                    nxt = (k + 1) % SLOTS

                    @pl.when(ci + 1 < NCHUNK)
                    def _prefetch():
                        @pl.when(ci >= 2)
                        def _drain():
                            wait_scatter(nxt)

                        fire_inputs(ci + 1, nxt)

                    wait_inputs(k)
                    y0 = row_base + ci * RCH
                    dx_cur = dx_r[k]
                    dy_cur = dy_r[k]
                    w_cur = w_r[k]
                    w1_cur = w1_r[k]
                    idx_cur = idx_r[k]
                    yfs = [(y0 + r).astype(jnp.float32) for r in range(RCH)]

                    def vec_body(j, rcarry):
                        xf = xf_buf[pl.ds(j * L, L)]
                        for r in range(RCH):
                            dxv = dx_cur[r, pl.ds(j * L, L)]
                            dyv = dy_cur[r, pl.ds(j * L, L)]
                            wv = w_cur[r, pl.ds(j * L, L)]
                            vx = (xf + R * dxv + MAGIC) - MAGIC
                            vy = (yfs[r] + R * dyv + MAGIC) - MAGIC
                            vx = jnp.minimum(jnp.maximum(vx, 0.0), W - 1.0)
                            vy = jnp.minimum(jnp.maximum(vy, 0.0), H - 1.0)
                            idx = (lax.shift_left(vy.astype(jnp.int32), 9)
                                   + vx.astype(jnp.int32) + hist_off)
                            o = r * W + j * L
                            idx_cur[pl.ds(o, L)] = idx
                            w1_cur[pl.ds(o, L)] = wv
                        return rcarry

                    lax.fori_loop(0, W // L, vec_body, 0)
                    pltpu.async_copy(
                        w1_cur, hist.at[idx_cur], ssc_r[k], add=True)

            return carry

        lax.fori_loop(0, NCHUNK, chunk_body, 0)
        for sl in range(SLOTS):
            wait_scatter(sl)
        plsc.subcore_barrier()

        @pl.when(pidx + 1 < NPASS)
        def _next_pass_prefetch():
            fire_inputs(0, 0, bg=b_global + PASS_BATCHES)

        pass_out = (c * BATCHES_PER_CORE + pidx * PASS_BATCHES) * P
        pltpu.sync_copy(hist.at[pl.ds(myslice, SLICE)],
                        out_hbm.at[pl.ds(pass_out + myslice, SLICE)])
        return pcarry

    lax.fori_loop(0, NPASS, pass_body, 0)


def kernel(stem_keypoint_output, stem_offset_output):
    w3 = stem_keypoint_output.reshape(B, H, W)
    off3 = stem_offset_output.reshape(2 * B, H, W)
    mesh = plsc.VectorSubcoreMesh(core_axis_name="c", subcore_axis_name="s")
    out = pl.kernel(
        _body,
        out_type=jax.ShapeDtypeStruct((B * P,), jnp.float32),
        mesh=mesh,
        compiler_params=pltpu.CompilerParams(use_tc_tiling_on_sc=True),
        scratch_types=[
            pltpu.VMEM_SHARED((HIST,), jnp.float32),
            pltpu.VMEM((RCH, W), jnp.float32),
            pltpu.VMEM((RCH, W), jnp.float32),
            pltpu.VMEM((RCH, W), jnp.float32),
            pltpu.VMEM((RCH, W), jnp.float32),
            pltpu.VMEM((RCH, W), jnp.float32),
            pltpu.VMEM((RCH, W), jnp.float32),
            pltpu.VMEM((RCH, W), jnp.float32),
            pltpu.VMEM((RCH, W), jnp.float32),
            pltpu.VMEM((RCH, W), jnp.float32),
            pltpu.VMEM((CH,), jnp.float32),
            pltpu.VMEM((CH,), jnp.float32),
            pltpu.VMEM((CH,), jnp.float32),
            pltpu.VMEM((CH,), jnp.int32),
            pltpu.VMEM((CH,), jnp.int32),
            pltpu.VMEM((CH,), jnp.int32),
            pltpu.VMEM((ZB,), jnp.float32),
            pltpu.VMEM((W,), jnp.float32),
            pltpu.SemaphoreType.DMA,
            pltpu.SemaphoreType.DMA,
            pltpu.SemaphoreType.DMA,
            pltpu.SemaphoreType.DMA,
            pltpu.SemaphoreType.DMA,
            pltpu.SemaphoreType.DMA,
        ],
    )(w3, off3)
    return out.reshape(B, H, W)
